# Initial kernel scaffold; baseline (speedup 1.0000x reference)
import jax
import jax.numpy as jnp
from jax.experimental import pallas as pl


def kernel(mask, mask_embed):
    raise NotImplementedError("write your pallas kernel here")


# speedup vs baseline: 181.1213x; 181.1213x over previous
import math

import jax
import jax.numpy as jnp
import numpy as np
from jax.experimental import pallas as pl

MAX_LEN = 500
EMBED_DIM = 128


def _pe_table() -> np.ndarray:
    pe = np.zeros((MAX_LEN, EMBED_DIM), dtype=np.float32)
    position = np.arange(0, MAX_LEN, dtype=np.float32)[:, None]
    div_term = np.exp(
        np.arange(0, EMBED_DIM, 2, dtype=np.float32) * (-math.log(10000.0) / EMBED_DIM)
    )
    pe[:, 0::2] = np.sin(position * div_term)
    pe[:, 1::2] = np.cos(position * div_term)
    return pe


def _body(mask_ref, emb_ref, pe_ref, out_ref):
    f = mask_ref.shape[-1]
    s = jnp.sum(mask_ref[...].astype(jnp.float32), axis=-1)
    e0 = emb_ref[0, :]
    de = (emb_ref[1, :] - e0) * (1.0 / f)
    base = pe_ref[...] + e0[None, :]
    out_ref[...] = base[None, :, :] + s[:, :, None] * de[None, None, :]


def kernel(mask, mask_embed):
    B, L, F = mask.shape
    D = mask_embed.shape[1]
    pe = jnp.asarray(_pe_table()[:L])

    BB = 64
    grid = (B // BB,)
    out = pl.pallas_call(
        _body,
        grid=grid,
        in_specs=[
            pl.BlockSpec((BB, L, F), lambda i: (i, 0, 0)),
            pl.BlockSpec((2, D), lambda i: (0, 0)),
            pl.BlockSpec((L, D), lambda i: (0, 0)),
        ],
        out_specs=pl.BlockSpec((BB, L, D), lambda i: (i, 0, 0)),
        out_shape=jax.ShapeDtypeStruct((B, L, D), jnp.float32),
    )(mask.astype(jnp.int32), mask_embed, pe)
    return out


# speedup vs baseline: 183.9969x; 1.0159x over previous
import math

import jax
import jax.numpy as jnp
import numpy as np
from jax.experimental import pallas as pl

MAX_LEN = 500
EMBED_DIM = 128


def _pe_table() -> np.ndarray:
    pe = np.zeros((MAX_LEN, EMBED_DIM), dtype=np.float32)
    position = np.arange(0, MAX_LEN, dtype=np.float32)[:, None]
    div_term = np.exp(
        np.arange(0, EMBED_DIM, 2, dtype=np.float32) * (-math.log(10000.0) / EMBED_DIM)
    )
    pe[:, 0::2] = np.sin(position * div_term)
    pe[:, 1::2] = np.cos(position * div_term)
    return pe


def _body(mask_ref, emb_ref, pe_ref, out_ref):
    bb, ll, f = mask_ref.shape
    d = out_ref.shape[-1]
    e0 = emb_ref[0, :]
    de = (emb_ref[1, :] - e0) * (1.0 / f)
    m = jnp.broadcast_to(de[None, :], (f, d))
    base = pe_ref[...] + e0[None, :]
    a = mask_ref[...].reshape(bb * ll, f).astype(jnp.float32)
    mm = jnp.dot(a, m, preferred_element_type=jnp.float32)
    out_ref[...] = mm.reshape(bb, ll, d) + base[None, :, :]


def kernel(mask, mask_embed):
    B, L, F = mask.shape
    D = mask_embed.shape[1]
    pe = jnp.asarray(_pe_table()[:L])

    BB = 64
    grid = (B // BB,)
    out = pl.pallas_call(
        _body,
        grid=grid,
        in_specs=[
            pl.BlockSpec((BB, L, F), lambda i: (i, 0, 0)),
            pl.BlockSpec((2, D), lambda i: (0, 0)),
            pl.BlockSpec((L, D), lambda i: (0, 0)),
        ],
        out_specs=pl.BlockSpec((BB, L, D), lambda i: (i, 0, 0)),
        out_shape=jax.ShapeDtypeStruct((B, L, D), jnp.float32),
    )(mask.astype(jnp.int32), mask_embed, pe)
    return out


# speedup vs baseline: 185.0190x; 1.0056x over previous
import math

import jax
import jax.numpy as jnp
import numpy as np
from jax.experimental import pallas as pl

MAX_LEN = 500
EMBED_DIM = 128


def _pe_table() -> np.ndarray:
    pe = np.zeros((MAX_LEN, EMBED_DIM), dtype=np.float32)
    position = np.arange(0, MAX_LEN, dtype=np.float32)[:, None]
    div_term = np.exp(
        np.arange(0, EMBED_DIM, 2, dtype=np.float32) * (-math.log(10000.0) / EMBED_DIM)
    )
    pe[:, 0::2] = np.sin(position * div_term)
    pe[:, 1::2] = np.cos(position * div_term)
    return pe


def _body(mask_ref, emb_ref, pe_ref, out_ref):
    bb, ll, f = mask_ref.shape
    d = out_ref.shape[-1]
    e0 = emb_ref[0, :]
    de = (emb_ref[1, :] - e0) * (1.0 / f)
    m = jnp.broadcast_to(de[None, :], (f, d))
    base = pe_ref[...] + e0[None, :]
    a = mask_ref[...].reshape(bb * ll, f).astype(jnp.float32)
    mm = jnp.dot(a, m, preferred_element_type=jnp.float32)
    out_ref[...] = mm.reshape(bb, ll, d) + base[None, :, :]


def kernel(mask, mask_embed):
    B, L, F = mask.shape
    D = mask_embed.shape[1]
    pe = jnp.asarray(_pe_table()[:L])

    BB = 128
    grid = (B // BB,)
    out = pl.pallas_call(
        _body,
        grid=grid,
        in_specs=[
            pl.BlockSpec((BB, L, F), lambda i: (i, 0, 0)),
            pl.BlockSpec((2, D), lambda i: (0, 0)),
            pl.BlockSpec((L, D), lambda i: (0, 0)),
        ],
        out_specs=pl.BlockSpec((BB, L, D), lambda i: (i, 0, 0)),
        out_shape=jax.ShapeDtypeStruct((B, L, D), jnp.float32),
    )(mask.astype(jnp.int32), mask_embed, pe)
    return out
